# baseline (speedup 1.0000x reference)
import functools

import jax
import jax.numpy as jnp
from jax import lax
from jax.experimental import pallas as pl
from jax.experimental.pallas import tpu as pltpu
from jax.experimental.pallas import tpu_sc as plsc

B, S, H, D = 4, 2048, 32, 128
NSEG = 2 * B
NC, NS = 2, 16
NW = NC * NS
C = 8
LANES = 16
DL = D // LANES


def _worker_params(range_ids):
    r = range_ids.astype(jnp.int32)
    starts = jnp.concatenate([r[:, 0], r[:, 2]])
    cnts = jnp.concatenate([r[:, 1] - r[:, 0] + 1, r[:, 3] - r[:, 2] + 1])
    off = jnp.cumsum(cnts) - cnts
    total = off[-1] + cnts[-1]
    q = (total + NW - 1) // NW
    w = jnp.arange(NW, dtype=jnp.int32)
    lo = jnp.minimum(w * q, total)[:, None]
    hi = jnp.minimum(lo + q, total)
    a = jnp.maximum(lo, off[None, :])
    bb = jnp.minimum(hi, (off + cnts)[None, :])
    n_ws = jnp.maximum(0, bb - a)
    rs = starts[None, :] + jnp.clip(a - off[None, :], 0, cnts[None, :])
    return jnp.stack([rs, n_ws], axis=-1).reshape(NW, 1, 16).astype(jnp.int32)


def _sc_partial_sums(pooled, params):
    mesh = plsc.VectorSubcoreMesh(core_axis_name="c", subcore_axis_name="s")

    @functools.partial(
        pl.kernel,
        mesh=mesh,
        out_type=jax.ShapeDtypeStruct((NW, NSEG * H, D), jnp.float32),
        scratch_types=[
            pltpu.VMEM((1, 16), jnp.int32),
            pltpu.VMEM((2, C, H, D), jnp.float32),
            pltpu.VMEM((NSEG * H, D), jnp.float32),
            pltpu.SemaphoreType.DMA,
            pltpu.SemaphoreType.DMA,
        ],
    )
    def sck(p_hbm, params_hbm, out_hbm, pv, buf2, acc, sem0, sem1):
        wid = lax.axis_index("c") * NS + lax.axis_index("s")
        pltpu.sync_copy(params_hbm.at[wid], pv)
        pvec = pv[0, :]
        zero16 = jnp.zeros((LANES,), jnp.float32)

        def zbody(i, _):
            for l in range(DL):
                acc[i, pl.ds(l * LANES, LANES)] = zero16
            return 0

        lax.fori_loop(0, NSEG * H, zbody, 0)

        for s8 in range(NSEG):
            bb = s8 % B
            row_start = pvec[2 * s8]
            n = pvec[2 * s8 + 1]
            lim = row_start + n
            first_blk = row_start // C
            n_blk = jnp.where(n > 0, (lim - 1) // C - first_blk + 1, 0)

            def start_dma(kk, first_blk=first_blk, bb=bb):
                base = (first_blk + kk) * C

                @pl.when(kk % 2 == 0)
                def _():
                    pltpu.async_copy(p_hbm.at[bb, pl.ds(base, C)],
                                     buf2.at[0], sem0)

                @pl.when(kk % 2 == 1)
                def _():
                    pltpu.async_copy(p_hbm.at[bb, pl.ds(base, C)],
                                     buf2.at[1], sem1)

            def wait_dma(kk, bb=bb):
                @pl.when(kk % 2 == 0)
                def _():
                    pltpu.make_async_copy(p_hbm.at[bb, pl.ds(0, C)],
                                          buf2.at[0], sem0).wait()

                @pl.when(kk % 2 == 1)
                def _():
                    pltpu.make_async_copy(p_hbm.at[bb, pl.ds(0, C)],
                                          buf2.at[1], sem1).wait()

            @pl.when(n_blk > 0)
            def _():
                start_dma(jnp.int32(0))

            def chunk(kk, _, row_start=row_start, lim=lim,
                      first_blk=first_blk, s8=s8, n_blk=n_blk):
                base = (first_blk + kk) * C

                @pl.when(kk + 1 < n_blk)
                def _():
                    start_dma(kk + 1)

                wait_dma(kk)
                cur = kk % 2
                ws = []
                for j in range(C):
                    inseg = (base + j >= row_start) & (base + j < lim)
                    ws.append(jnp.where(inseg, 1.0, 0.0).astype(jnp.float32))

                def hloop(h, _):
                    arow = s8 * H + h
                    for l in range(DL):
                        sl = pl.ds(l * LANES, LANES)
                        t = [buf2[cur, j, h, sl] * ws[j] for j in range(C)]
                        while len(t) > 1:
                            t = [t[i] + t[i + 1] for i in range(0, len(t), 2)]
                        acc[arow, sl] = acc[arow, sl] + t[0]
                    return 0

                lax.fori_loop(0, H, hloop, 0)
                return 0

            lax.fori_loop(0, n_blk, chunk, 0)

        pltpu.sync_copy(acc, out_hbm.at[wid])

    return sck(pooled, params)


def _tc_head(partials, range_ids, W1, W2, W3, b1r, b2r, b3r, ltr):
    def body(p_ref, rid_ref, w1_ref, w2_ref, w3_ref, b1_ref, b2_ref, b3_ref,
             lt_ref, zs_ref, zh_ref, lg_ref):
        red = p_ref[0]
        for i in range(1, NW):
            red = red + p_ref[i]
        xs = []
        for b in range(B):
            c0 = (rid_ref[b, 1] - rid_ref[b, 0] + 1).astype(jnp.float32)
            c1 = (rid_ref[b, 3] - rid_ref[b, 2] + 1).astype(jnp.float32)
            ctx = lax.slice(red, (b * H, 0), ((b + 1) * H, D))
            qq = lax.slice(red, ((B + b) * H, 0), ((B + b + 1) * H, D))
            xs.append(ctx * (0.5 / c0) + qq * (0.5 / c1))
        x = jnp.concatenate(xs, axis=0)

        def silu(t):
            return t / (1.0 + jnp.exp(-t))

        dn = (((1,), (1,)), ((), ()))
        h = silu(lax.dot_general(x, w1_ref[...], dn) + b1_ref[...])
        h = silu(lax.dot_general(h, w2_ref[...], dn) + b2_ref[...])
        lg = lax.dot_general(h, w3_ref[...], dn) + b3_ref[...]
        tau = jnp.exp(lt_ref[0, 0])
        bl = lax.slice(lg, (0, 1), (B * H, 2)) - lax.slice(lg, (0, 0), (B * H, 1))
        zs = 1.0 / (1.0 + jnp.exp(-bl / tau))
        zs_ref[...] = zs
        zh_ref[...] = jnp.where(zs > 0.5, 1.0, 0.0)
        lg_ref[...] = lg

    f32 = jnp.float32
    return pl.pallas_call(
        body,
        out_shape=(
            jax.ShapeDtypeStruct((B * H, 1), f32),
            jax.ShapeDtypeStruct((B * H, 1), f32),
            jax.ShapeDtypeStruct((B * H, 2), f32),
        ),
        in_specs=[
            pl.BlockSpec(memory_space=pltpu.VMEM),
            pl.BlockSpec(memory_space=pltpu.SMEM),
            pl.BlockSpec(memory_space=pltpu.VMEM),
            pl.BlockSpec(memory_space=pltpu.VMEM),
            pl.BlockSpec(memory_space=pltpu.VMEM),
            pl.BlockSpec(memory_space=pltpu.VMEM),
            pl.BlockSpec(memory_space=pltpu.VMEM),
            pl.BlockSpec(memory_space=pltpu.VMEM),
            pl.BlockSpec(memory_space=pltpu.SMEM),
        ],
    )(partials, range_ids, W1, W2, W3, b1r, b2r, b3r, ltr)


def kernel(pooled_input, range_ids, W1, b1, W2, b2, W3, b3, log_temp):
    params = _worker_params(range_ids)
    partials = _sc_partial_sums(pooled_input, params)
    zs, zh, lg = _tc_head(
        partials, range_ids.astype(jnp.int32), W1, W2, W3,
        b1.reshape(1, 256), b2.reshape(1, 128), b3.reshape(1, 2),
        log_temp.reshape(1, 1),
    )
    z_soft = zs.reshape(B, H)
    z_hard = zh.reshape(B, H)
    logits = lg.reshape(B, H, 2)
    return (z_soft, z_hard, z_hard, logits)


# speedup vs baseline: 1.2453x; 1.2453x over previous
import functools

import jax
import jax.numpy as jnp
from jax import lax
from jax.experimental import pallas as pl
from jax.experimental.pallas import tpu as pltpu
from jax.experimental.pallas import tpu_sc as plsc

B, S, H, D = 4, 2048, 32, 128
NSEG = 2 * B
NC, NS = 2, 16
NW = NC * NS
C = 8
LANES = 16
DL = D // LANES


def _worker_params(range_ids):
    r = range_ids.astype(jnp.int32)
    starts = jnp.concatenate([r[:, 0], r[:, 2]])
    cnts = jnp.concatenate([r[:, 1] - r[:, 0] + 1, r[:, 3] - r[:, 2] + 1])
    off = jnp.cumsum(cnts) - cnts
    total = off[-1] + cnts[-1]
    q = (total + NW - 1) // NW
    w = jnp.arange(NW, dtype=jnp.int32)
    lo = jnp.minimum(w * q, total)[:, None]
    hi = jnp.minimum(lo + q, total)
    a = jnp.maximum(lo, off[None, :])
    bb = jnp.minimum(hi, (off + cnts)[None, :])
    n_ws = jnp.maximum(0, bb - a)
    rs = starts[None, :] + jnp.clip(a - off[None, :], 0, cnts[None, :])
    return jnp.stack([rs, n_ws], axis=-1).reshape(NW, 1, 16).astype(jnp.int32)


def _sc_partial_sums(pooled, params):
    mesh = plsc.VectorSubcoreMesh(core_axis_name="c", subcore_axis_name="s")

    @functools.partial(
        pl.kernel,
        mesh=mesh,
        out_type=jax.ShapeDtypeStruct((NW, NSEG * H, D), jnp.float32),
        scratch_types=[
            pltpu.VMEM((1, 16), jnp.int32),
            pltpu.VMEM((2, C, H, D), jnp.float32),
            pltpu.VMEM((NSEG * H, D), jnp.float32),
            pltpu.SemaphoreType.DMA,
            pltpu.SemaphoreType.DMA,
        ],
    )
    def sck(p_hbm, params_hbm, out_hbm, pv, buf2, acc, sem0, sem1):
        wid = lax.axis_index("c") * NS + lax.axis_index("s")
        pltpu.sync_copy(params_hbm.at[wid], pv)
        pvec = pv[0, :]
        zero16 = jnp.zeros((LANES,), jnp.float32)

        @plsc.parallel_loop(0, NSEG * H, unroll=4)
        def _(i):
            for l in range(DL):
                acc[i, pl.ds(l * LANES, LANES)] = zero16

        for s8 in range(NSEG):
            bb = s8 % B
            row_start = pvec[2 * s8]
            n = pvec[2 * s8 + 1]
            lim = row_start + n
            first_blk = row_start // C
            n_blk = jnp.where(n > 0, (lim - 1) // C - first_blk + 1, 0)

            def start_dma(kk, first_blk=first_blk, bb=bb):
                base = (first_blk + kk) * C

                @pl.when(kk % 2 == 0)
                def _():
                    pltpu.async_copy(p_hbm.at[bb, pl.ds(base, C)],
                                     buf2.at[0], sem0)

                @pl.when(kk % 2 == 1)
                def _():
                    pltpu.async_copy(p_hbm.at[bb, pl.ds(base, C)],
                                     buf2.at[1], sem1)

            def wait_dma(kk, bb=bb):
                @pl.when(kk % 2 == 0)
                def _():
                    pltpu.make_async_copy(p_hbm.at[bb, pl.ds(0, C)],
                                          buf2.at[0], sem0).wait()

                @pl.when(kk % 2 == 1)
                def _():
                    pltpu.make_async_copy(p_hbm.at[bb, pl.ds(0, C)],
                                          buf2.at[1], sem1).wait()

            @pl.when(n_blk > 0)
            def _():
                start_dma(jnp.int32(0))

            def chunk(kk, _, row_start=row_start, lim=lim,
                      first_blk=first_blk, s8=s8, n_blk=n_blk):
                base = (first_blk + kk) * C

                @pl.when(kk + 1 < n_blk)
                def _():
                    start_dma(kk + 1)

                wait_dma(kk)
                cur = kk % 2
                ws = []
                for j in range(C):
                    inseg = (base + j >= row_start) & (base + j < lim)
                    ws.append(jnp.where(inseg, 1.0, 0.0).astype(jnp.float32))

                @plsc.parallel_loop(0, H, unroll=2)
                def _(h):
                    arow = s8 * H + h
                    for l in range(DL):
                        sl = pl.ds(l * LANES, LANES)
                        t = [buf2[cur, j, h, sl] * ws[j] for j in range(C)]
                        while len(t) > 1:
                            t = [t[i] + t[i + 1] for i in range(0, len(t), 2)]
                        plsc.addupdate(acc.at[arow, sl], t[0])

                return 0

            lax.fori_loop(0, n_blk, chunk, 0)

        pltpu.sync_copy(acc, out_hbm.at[wid])

    return sck(pooled, params)


def _tc_head(partials, range_ids, W1, W2, W3, b1r, b2r, b3r, ltr):
    def body(p_ref, rid_ref, w1_ref, w2_ref, w3_ref, b1_ref, b2_ref, b3_ref,
             lt_ref, zs_ref, zh_ref, lg_ref):
        red = p_ref[0]
        for i in range(1, NW):
            red = red + p_ref[i]
        xs = []
        for b in range(B):
            c0 = (rid_ref[b, 1] - rid_ref[b, 0] + 1).astype(jnp.float32)
            c1 = (rid_ref[b, 3] - rid_ref[b, 2] + 1).astype(jnp.float32)
            ctx = lax.slice(red, (b * H, 0), ((b + 1) * H, D))
            qq = lax.slice(red, ((B + b) * H, 0), ((B + b + 1) * H, D))
            xs.append(ctx * (0.5 / c0) + qq * (0.5 / c1))
        x = jnp.concatenate(xs, axis=0)

        def silu(t):
            return t / (1.0 + jnp.exp(-t))

        dn = (((1,), (1,)), ((), ()))
        h = silu(lax.dot_general(x, w1_ref[...], dn) + b1_ref[...])
        h = silu(lax.dot_general(h, w2_ref[...], dn) + b2_ref[...])
        lg = lax.dot_general(h, w3_ref[...], dn) + b3_ref[...]
        tau = jnp.exp(lt_ref[0, 0])
        bl = lax.slice(lg, (0, 1), (B * H, 2)) - lax.slice(lg, (0, 0), (B * H, 1))
        zs = 1.0 / (1.0 + jnp.exp(-bl / tau))
        zs_ref[...] = zs
        zh_ref[...] = jnp.where(zs > 0.5, 1.0, 0.0)
        lg_ref[...] = lg

    f32 = jnp.float32
    return pl.pallas_call(
        body,
        out_shape=(
            jax.ShapeDtypeStruct((B * H, 1), f32),
            jax.ShapeDtypeStruct((B * H, 1), f32),
            jax.ShapeDtypeStruct((B * H, 2), f32),
        ),
        in_specs=[
            pl.BlockSpec(memory_space=pltpu.VMEM),
            pl.BlockSpec(memory_space=pltpu.SMEM),
            pl.BlockSpec(memory_space=pltpu.VMEM),
            pl.BlockSpec(memory_space=pltpu.VMEM),
            pl.BlockSpec(memory_space=pltpu.VMEM),
            pl.BlockSpec(memory_space=pltpu.VMEM),
            pl.BlockSpec(memory_space=pltpu.VMEM),
            pl.BlockSpec(memory_space=pltpu.VMEM),
            pl.BlockSpec(memory_space=pltpu.SMEM),
        ],
    )(partials, range_ids, W1, W2, W3, b1r, b2r, b3r, ltr)


def kernel(pooled_input, range_ids, W1, b1, W2, b2, W3, b3, log_temp):
    params = _worker_params(range_ids)
    partials = _sc_partial_sums(pooled_input, params)
    zs, zh, lg = _tc_head(
        partials, range_ids.astype(jnp.int32), W1, W2, W3,
        b1.reshape(1, 256), b2.reshape(1, 128), b3.reshape(1, 2),
        log_temp.reshape(1, 1),
    )
    z_soft = zs.reshape(B, H)
    z_hard = zh.reshape(B, H)
    logits = lg.reshape(B, H, 2)
    return (z_soft, z_hard, z_hard, logits)


# speedup vs baseline: 1.4215x; 1.1415x over previous
import functools

import jax
import jax.numpy as jnp
from jax import lax
from jax.experimental import pallas as pl
from jax.experimental.pallas import tpu as pltpu
from jax.experimental.pallas import tpu_sc as plsc

B, S, H, D = 4, 2048, 32, 128
NSEG = 2 * B
NC, NS = 2, 16
NW = NC * NS
C = 8
LANES = 16
DL = D // LANES


def _worker_params(range_ids):
    r = range_ids.astype(jnp.int32)
    starts = jnp.concatenate([r[:, 0], r[:, 2]])
    cnts = jnp.concatenate([r[:, 1] - r[:, 0] + 1, r[:, 3] - r[:, 2] + 1])
    off = jnp.cumsum(cnts) - cnts
    total = off[-1] + cnts[-1]
    q = (total + NW - 1) // NW
    w = jnp.arange(NW, dtype=jnp.int32)
    lo = jnp.minimum(w * q, total)[:, None]
    hi = jnp.minimum(lo + q, total)
    a = jnp.maximum(lo, off[None, :])
    bb = jnp.minimum(hi, (off + cnts)[None, :])
    n_ws = jnp.maximum(0, bb - a)
    rs = starts[None, :] + jnp.clip(a - off[None, :], 0, cnts[None, :])
    return jnp.stack([rs, n_ws], axis=-1).reshape(NW, 1, 16).astype(jnp.int32)


CH = C * H
TRASH = NSEG * H


def _sc_partial_sums(p2, params):
    mesh = plsc.VectorSubcoreMesh(core_axis_name="c", subcore_axis_name="s")

    @functools.partial(
        pl.kernel,
        mesh=mesh,
        out_type=jax.ShapeDtypeStruct((NC, NSEG * H, D), jnp.float32),
        scratch_types=[
            pltpu.VMEM((1, 16), jnp.int32),
            pltpu.VMEM((2, CH, D), jnp.float32),
            pltpu.VMEM((2, 128), jnp.int32),
            pltpu.VMEM((16, D), jnp.float32),
            pltpu.VMEM_SHARED((264, D), jnp.float32),
            pltpu.SemaphoreType.DMA,
            pltpu.SemaphoreType.DMA,
        ],
    )
    def sck(p_hbm, params_hbm, out_hbm, pv, buf2, idxv, zbuf, shared,
            sem0, sem1):
        cc = lax.axis_index("c")
        sid = lax.axis_index("s")
        wid = cc * NS + sid
        pltpu.sync_copy(params_hbm.at[wid], pv)
        pvec = pv[0, :]
        zero16 = jnp.zeros((LANES,), jnp.float32)
        iota = lax.iota(jnp.int32, LANES)

        for rr in range(16):
            for l in range(DL):
                zbuf[rr, pl.ds(l * LANES, LANES)] = zero16
        pltpu.sync_copy(zbuf, shared.at[pl.ds(sid * 16, 16)])
        plsc.subcore_barrier()

        for s8 in range(NSEG):
            bb = s8 % B
            row_start = pvec[2 * s8]
            n = pvec[2 * s8 + 1]
            lim = row_start + n
            nc_ = (n + C - 1) // C

            def pos_of(kk, row_start=row_start):
                return jnp.minimum(row_start + kk * C, S - C)

            def start_dma(kk, bb=bb, pos_of=pos_of):
                pos = pos_of(kk)

                @pl.when(kk % 2 == 0)
                def _():
                    pltpu.async_copy(p_hbm.at[bb, pl.ds(pos * H, CH)],
                                     buf2.at[0], sem0)

                @pl.when(kk % 2 == 1)
                def _():
                    pltpu.async_copy(p_hbm.at[bb, pl.ds(pos * H, CH)],
                                     buf2.at[1], sem1)

            def wait_dma(kk, bb=bb):
                @pl.when(kk % 2 == 0)
                def _():
                    pltpu.make_async_copy(p_hbm.at[bb, pl.ds(0, CH)],
                                          buf2.at[0], sem0).wait()

                @pl.when(kk % 2 == 1)
                def _():
                    pltpu.make_async_copy(p_hbm.at[bb, pl.ds(0, CH)],
                                          buf2.at[1], sem1).wait()

            @pl.when(nc_ > 0)
            def _():
                start_dma(jnp.int32(0))

            def chunk(kk, _, row_start=row_start, lim=lim, s8=s8,
                      nc_=nc_, pos_of=pos_of):
                @pl.when(kk + 1 < nc_)
                def _():
                    start_dma(kk + 1)

                pos = pos_of(kk)
                start0 = row_start + kk * C
                d = start0 - pos
                v = jnp.minimum(lim - start0, C)
                for m in range(16):
                    jm = m // 2
                    dest = s8 * H + (m % 2) * LANES + iota
                    cond = (jm >= d) & (jm < d + v)
                    idx_blk = jnp.where(cond, dest, TRASH)
                    idxv[m // 8, pl.ds((m % 8) * LANES, LANES)] = idx_blk

                wait_dma(kk)
                cur = kk % 2
                pltpu.sync_copy(buf2.at[cur, pl.ds(0, 128)],
                                shared.at[idxv.at[0]], add=True)
                pltpu.sync_copy(buf2.at[cur, pl.ds(128, 128)],
                                shared.at[idxv.at[1]], add=True)
                return 0

            lax.fori_loop(0, nc_, chunk, 0)

        plsc.subcore_barrier()
        pltpu.sync_copy(shared.at[pl.ds(sid * 16, 16)],
                        out_hbm.at[cc, pl.ds(sid * 16, 16)])

    return sck(p2, params)


def _tc_head(partials, range_ids, W1, W2, W3, b1r, b2r, b3r, ltr):
    def body(p_ref, rid_ref, w1_ref, w2_ref, w3_ref, b1_ref, b2_ref, b3_ref,
             lt_ref, zs_ref, zh_ref, lg_ref):
        red = p_ref[0]
        for i in range(1, NC):
            red = red + p_ref[i]
        xs = []
        for b in range(B):
            c0 = (rid_ref[b, 1] - rid_ref[b, 0] + 1).astype(jnp.float32)
            c1 = (rid_ref[b, 3] - rid_ref[b, 2] + 1).astype(jnp.float32)
            ctx = lax.slice(red, (b * H, 0), ((b + 1) * H, D))
            qq = lax.slice(red, ((B + b) * H, 0), ((B + b + 1) * H, D))
            xs.append(ctx * (0.5 / c0) + qq * (0.5 / c1))
        x = jnp.concatenate(xs, axis=0)

        def silu(t):
            return t / (1.0 + jnp.exp(-t))

        dn = (((1,), (1,)), ((), ()))
        h = silu(lax.dot_general(x, w1_ref[...], dn) + b1_ref[...])
        h = silu(lax.dot_general(h, w2_ref[...], dn) + b2_ref[...])
        lg = lax.dot_general(h, w3_ref[...], dn) + b3_ref[...]
        tau = jnp.exp(lt_ref[0, 0])
        bl = lax.slice(lg, (0, 1), (B * H, 2)) - lax.slice(lg, (0, 0), (B * H, 1))
        zs = 1.0 / (1.0 + jnp.exp(-bl / tau))
        zs_ref[...] = zs
        zh_ref[...] = jnp.where(zs > 0.5, 1.0, 0.0)
        lg_ref[...] = lg

    f32 = jnp.float32
    return pl.pallas_call(
        body,
        out_shape=(
            jax.ShapeDtypeStruct((B * H, 1), f32),
            jax.ShapeDtypeStruct((B * H, 1), f32),
            jax.ShapeDtypeStruct((B * H, 2), f32),
        ),
        in_specs=[
            pl.BlockSpec(memory_space=pltpu.VMEM),
            pl.BlockSpec(memory_space=pltpu.SMEM),
            pl.BlockSpec(memory_space=pltpu.VMEM),
            pl.BlockSpec(memory_space=pltpu.VMEM),
            pl.BlockSpec(memory_space=pltpu.VMEM),
            pl.BlockSpec(memory_space=pltpu.VMEM),
            pl.BlockSpec(memory_space=pltpu.VMEM),
            pl.BlockSpec(memory_space=pltpu.VMEM),
            pl.BlockSpec(memory_space=pltpu.SMEM),
        ],
    )(partials, range_ids, W1, W2, W3, b1r, b2r, b3r, ltr)


def kernel(pooled_input, range_ids, W1, b1, W2, b2, W3, b3, log_temp):
    params = _worker_params(range_ids)
    p2 = pooled_input.reshape(B, S * H, D)
    partials = _sc_partial_sums(p2, params)
    zs, zh, lg = _tc_head(
        partials, range_ids.astype(jnp.int32), W1, W2, W3,
        b1.reshape(1, 256), b2.reshape(1, 128), b3.reshape(1, 2),
        log_temp.reshape(1, 1),
    )
    z_soft = zs.reshape(B, H)
    z_hard = zh.reshape(B, H)
    logits = lg.reshape(B, H, 2)
    return (z_soft, z_hard, z_hard, logits)
